# baseline (speedup 1.0000x reference)
import functools

import jax
import jax.numpy as jnp
from jax import lax
from jax.experimental import pallas as pl
from jax.experimental.pallas import tpu as pltpu
from jax.experimental.pallas import tpu_sc as plsc

N_NODE = 50000
E = 160000
D = 128
G = 32
NG = D // G
CONV = 16

NS = 16
K = 128
BPW = 80
HB = BPW // 2
EPW = K * BPW
EPAD = NS * EPW
NPAD = 50048
RPW = NPAD // NS
TRASH = N_NODE

_f32 = jnp.float32


def _tables_body(hu, uc, wp, bp, wc, bc, w1, w2, bv, lg, lb, *outs):
    x = hu[...]
    tp = jnp.dot(x, wp[...], preferred_element_type=_f32) + bp[...]
    tc = (jnp.dot(x, wc[...], preferred_element_type=_f32) + bc[...]) * 0.7
    z = (jnp.dot(x, w1[...], preferred_element_type=_f32)
         + jnp.dot(uc[...], w2[...], preferred_element_type=_f32) + bv[...])
    mu = jnp.mean(z, axis=-1, keepdims=True)
    var = jnp.mean((z - mu) ** 2, axis=-1, keepdims=True)
    nm = jnp.maximum((z - mu) / jnp.sqrt(var + 1e-5) * lg[...] + lb[...], 0.0)
    for g in range(NG):
        outs[g][...] = tp[:, G * g:G * g + G]
        outs[NG + g][...] = tc[:, G * g:G * g + G]
        outs[2 * NG + g][...] = nm[:, G * g:G * g + G]


def _node_tables(h_user, user_context, W_pub, b_pub, W_com, b_com,
                 W1, W2, b_conv, ln_g, ln_b):
    blk = 1000
    grid = (N_NODE // blk,)
    full = lambda r, c: pl.BlockSpec((r, c), lambda i: (0, 0))
    return pl.pallas_call(
        _tables_body,
        grid=grid,
        in_specs=[
            pl.BlockSpec((blk, D), lambda i: (i, 0)),
            pl.BlockSpec((blk, CONV), lambda i: (i, 0)),
            full(D, D), full(1, D), full(D, D), full(1, D),
            full(D, D), full(CONV, D), full(1, D), full(1, D), full(1, D),
        ],
        out_specs=[pl.BlockSpec((blk, G), lambda i: (i, 0))] * (3 * NG),
        out_shape=[jax.ShapeDtypeStruct((N_NODE, G), _f32)] * (3 * NG),
    )(h_user, user_context, W_pub, b_pub.reshape(1, D), W_com,
      b_com.reshape(1, D), W1, W2, b_conv.reshape(1, D),
      ln_g.reshape(1, D), ln_b.reshape(1, D))


def _sc_body(*refs):
    (tbls, ef32, sp, dp, sc_, dc_, su, du, osum,
     acc, idxg, idxs, r0, r1, r2, cbuf, g0, g1, g2, s0, s1, s2, ss) = (
        refs[0:12], refs[12], refs[13], refs[14], refs[15], refs[16],
        refs[17], refs[18], refs[19:34], refs[34], refs[35], refs[36],
        refs[37], refs[38], refs[39], refs[40], refs[41], refs[42],
        refs[43], refs[44], refs[45], refs[46], refs[47])

    cid = lax.axis_index("c")
    sid = lax.axis_index("s")
    bufs = (r0, r1, r2)
    gsems = (g0, g1, g2)
    ssems = (s0, s1, s2)

    z16 = jnp.zeros((16,), _f32)
    one16 = jnp.where(lax.iota(jnp.int32, 16) == 0, 1.0, 0.0).astype(_f32)

    @pl.loop(0, K)
    def _(i):
        cbuf[i, pl.ds(0, 16)] = one16
        cbuf[i, pl.ds(16, 16)] = z16

    def one_pass(gather_tbl, seq_tbl, src_hbm, dst_hbm, out_hbm):
        @pl.loop(0, K)
        def _(i):
            r0[i, pl.ds(0, 16)] = z16
            r0[i, pl.ds(16, 16)] = z16
        base = sid * RPW
        for k in range(RPW // K):
            pltpu.sync_copy(r0, acc.at[pl.ds(base + k * K, K)])
        pltpu.sync_copy(r0.at[pl.ds(0, RPW % K)],
                        acc.at[pl.ds(base + (RPW // K) * K, RPW % K)])
        plsc.subcore_barrier()

        for h in range(BPW // HB):
            pltpu.sync_copy(dst_hbm.at[sid, pl.ds(h * HB, HB)], idxs)
            if gather_tbl is not None:
                pltpu.sync_copy(src_hbm.at[sid, pl.ds(h * HB, HB)], idxg)

            if gather_tbl is None and seq_tbl is None:
                @pl.loop(0, HB)
                def _(j):
                    pltpu.async_copy(cbuf, acc.at[idxs.at[j]], ss, add=True)

                @pl.loop(0, HB)
                def _(j):
                    pltpu.make_async_copy(cbuf, acc.at[idxs.at[j]], ss).wait()
            else:
                seq0 = sid * EPW + h * HB * K

                def g_desc(j, b):
                    if gather_tbl is not None:
                        return (gather_tbl.at[idxg.at[j]], bufs[b], gsems[b])
                    return (seq_tbl.at[pl.ds(seq0 + j * K, K)],
                            bufs[b], gsems[b])

                def s_desc(j, b):
                    return (bufs[b], acc.at[idxs.at[j]], ssems[b])

                pltpu.async_copy(*g_desc(0, 0))
                pltpu.async_copy(*g_desc(1, 1))

                @pl.loop(0, HB, step=3)
                def _(j0):
                    for b in range(3):
                        j = j0 + b

                        @pl.when(j < HB)
                        def _(j=j, b=b, g_desc=g_desc, s_desc=s_desc):
                            pltpu.make_async_copy(*g_desc(j, b)).wait()
                            pltpu.async_copy(*s_desc(j, b), add=True)

                            @pl.when(j >= 1)
                            def _(j=j, b=b, s_desc=s_desc):
                                pltpu.make_async_copy(
                                    *s_desc(j - 1, (b + 2) % 3)).wait()

                            @pl.when(j + 2 < HB)
                            def _(j=j, b=b, g_desc=g_desc):
                                pltpu.async_copy(*g_desc(j + 2, (b + 2) % 3))

                pltpu.make_async_copy(*s_desc(HB - 1, (HB - 1) % 3)).wait()

        plsc.subcore_barrier()
        pltpu.sync_copy(acc.at[pl.ds(sid * RPW, RPW)],
                        out_hbm.at[pl.ds(sid * RPW, RPW)])
        plsc.subcore_barrier()

    passes = []
    for g in range(NG):
        owner = 0 if g < 2 else 1
        passes.append((tbls[g], None, sp, dp, osum[g], owner))
        passes.append((tbls[NG + g], None, sc_, dc_, osum[NG + g], owner))
        passes.append((tbls[2 * NG + g], None, su, du, osum[2 * NG + g], owner))
    passes.append((None, ef32, None, dc_, osum[12], 0))
    passes.append((None, None, None, dp, osum[13], 1))
    passes.append((None, None, None, du, osum[14], 1))

    for gt, st, s_h, d_h, o_h, owner in passes:
        @pl.when(cid == owner)
        def _(gt=gt, st=st, s_h=s_h, d_h=d_h, o_h=o_h):
            one_pass(gt, st, s_h, d_h, o_h)


def _sc_segment_sums(tables, ef32, sp, dp, sc_, dc_, su, du):
    mesh = plsc.VectorSubcoreMesh(core_axis_name="c", subcore_axis_name="s")
    kern = pl.kernel(
        _sc_body,
        out_type=[jax.ShapeDtypeStruct((NPAD, G), _f32)] * 15,
        mesh=mesh,
        compiler_params=pltpu.CompilerParams(use_tc_tiling_on_sc=False),
        scratch_types=[
            pltpu.VMEM_SHARED((NPAD, G), _f32),
            pltpu.VMEM((HB, K), jnp.int32),
            pltpu.VMEM((HB, K), jnp.int32),
            pltpu.VMEM((K, G), _f32),
            pltpu.VMEM((K, G), _f32),
            pltpu.VMEM((K, G), _f32),
            pltpu.VMEM((K, G), _f32),
            pltpu.SemaphoreType.DMA,
            pltpu.SemaphoreType.DMA,
            pltpu.SemaphoreType.DMA,
            pltpu.SemaphoreType.DMA,
            pltpu.SemaphoreType.DMA,
            pltpu.SemaphoreType.DMA,
            pltpu.SemaphoreType.DMA,
        ],
    )
    return kern(*tables, ef32, sp, dp, sc_, dc_, su, du)


def _fin_body(wep, bep, *refs):
    sums = refs[0:12]
    efs, cp, cu = refs[12], refs[13], refs[14]
    pub, com, ucu = refs[15], refs[16], refs[17]
    cnt_c = efs[...][:, CONV:CONV + 1]
    invp = 1.0 / jnp.maximum(cp[...][:, 0:1], 1.0)
    invc = 1.0 / jnp.maximum(cnt_c, 1.0)
    invu = 1.0 / jnp.maximum(cu[...][:, 0:1], 1.0)
    pub[...] = jnp.concatenate([sums[g][...] for g in range(NG)], axis=1) * invp
    nonzero = jnp.where(cnt_c >= 1.0, 0.3, 0.0)
    base = (jnp.dot(efs[...][:, 0:CONV] * invc, wep[...],
                    preferred_element_type=_f32) + bep[...]) * nonzero
    com[...] = jnp.concatenate([sums[NG + g][...] for g in range(NG)],
                               axis=1) * invc + base
    ucu[...] = jnp.concatenate([sums[2 * NG + g][...] for g in range(NG)],
                               axis=1) * invu


def _finalize(W_ep, b_ep, sums):
    blk = 1000
    grid = (N_NODE // blk,)
    return pl.pallas_call(
        _fin_body,
        grid=grid,
        in_specs=[pl.BlockSpec((CONV, D), lambda i: (0, 0)),
                  pl.BlockSpec((1, D), lambda i: (0, 0))] +
                 [pl.BlockSpec((blk, G), lambda i: (i, 0))] * 15,
        out_specs=[pl.BlockSpec((blk, D), lambda i: (i, 0))] * 3,
        out_shape=[jax.ShapeDtypeStruct((N_NODE, D), _f32)] * 3,
    )(W_ep, b_ep.reshape(1, D), *sums)


@jax.jit
def kernel(h_user, h_post, user_context, edge_feat_comment, W_pub, b_pub,
           W_com, b_com, W_conv, b_conv, ln_g, ln_b, W_ep, b_ep,
           edge_index_publish, edge_index_comment, edge_index_ucu):
    tables = _node_tables(h_user, user_context, W_pub, b_pub, W_com, b_com,
                          W_conv[:D], W_conv[D:], b_conv, ln_g, ln_b)
    ef32 = jnp.concatenate(
        [edge_feat_comment, jnp.ones((E, 1), _f32),
         jnp.zeros((E, G - CONV - 1), _f32)], axis=1)
    ef32 = jnp.concatenate([ef32, jnp.zeros((EPAD - E, G), _f32)], axis=0)
    i32 = jnp.int32
    zpad = jnp.zeros((EPAD - E,), i32)
    tpad = jnp.full((EPAD - E,), TRASH, i32)

    def _src(row):
        return jnp.concatenate([row.astype(i32), zpad]).reshape(NS, BPW, K)

    def _dst(row):
        return jnp.concatenate([row.astype(i32), tpad]).reshape(NS, BPW, K)

    sp, dp = _src(edge_index_publish[0]), _dst(edge_index_publish[1])
    sc_, dc_ = _src(edge_index_comment[0]), _dst(edge_index_comment[1])
    su, du = _src(edge_index_ucu[0]), _dst(edge_index_ucu[1])
    sums = _sc_segment_sums(tables, ef32, sp, dp, sc_, dc_, su, du)
    pub, com, ucu = _finalize(W_ep, b_ep, sums)
    return (pub, com, ucu)


# speedup vs baseline: 1.0033x; 1.0033x over previous
import functools

import jax
import jax.numpy as jnp
from jax import lax
from jax.experimental import pallas as pl
from jax.experimental.pallas import tpu as pltpu
from jax.experimental.pallas import tpu_sc as plsc

N_NODE = 50000
E = 160000
D = 128
G = 32
NG = D // G
CONV = 16

NS = 16
K = 128
BPW = 80
HB = BPW // 2
EPW = K * BPW
EPAD = NS * EPW
NPAD = 50048
RPW = NPAD // NS
TRASH = N_NODE

_f32 = jnp.float32


def _tables_body(hu, uc, wp, bp, wc, bc, w1, w2, bv, lg, lb, *outs):
    x = hu[...]
    tp = jnp.dot(x, wp[...], preferred_element_type=_f32) + bp[...]
    tc = (jnp.dot(x, wc[...], preferred_element_type=_f32) + bc[...]) * 0.7
    z = (jnp.dot(x, w1[...], preferred_element_type=_f32)
         + jnp.dot(uc[...], w2[...], preferred_element_type=_f32) + bv[...])
    mu = jnp.mean(z, axis=-1, keepdims=True)
    var = jnp.mean((z - mu) ** 2, axis=-1, keepdims=True)
    nm = jnp.maximum((z - mu) / jnp.sqrt(var + 1e-5) * lg[...] + lb[...], 0.0)
    for g in range(NG):
        outs[g][...] = tp[:, G * g:G * g + G]
        outs[NG + g][...] = tc[:, G * g:G * g + G]
        outs[2 * NG + g][...] = nm[:, G * g:G * g + G]


def _node_tables(h_user, user_context, W_pub, b_pub, W_com, b_com,
                 W1, W2, b_conv, ln_g, ln_b):
    blk = 1000
    grid = (N_NODE // blk,)
    full = lambda r, c: pl.BlockSpec((r, c), lambda i: (0, 0))
    return pl.pallas_call(
        _tables_body,
        grid=grid,
        in_specs=[
            pl.BlockSpec((blk, D), lambda i: (i, 0)),
            pl.BlockSpec((blk, CONV), lambda i: (i, 0)),
            full(D, D), full(1, D), full(D, D), full(1, D),
            full(D, D), full(CONV, D), full(1, D), full(1, D), full(1, D),
        ],
        out_specs=[pl.BlockSpec((blk, G), lambda i: (i, 0))] * (3 * NG),
        out_shape=[jax.ShapeDtypeStruct((N_NODE, G), _f32)] * (3 * NG),
    )(h_user, user_context, W_pub, b_pub.reshape(1, D), W_com,
      b_com.reshape(1, D), W1, W2, b_conv.reshape(1, D),
      ln_g.reshape(1, D), ln_b.reshape(1, D))


def _sc_body(*refs):
    (tbls, ef32, sp, dp, sc_, dc_, su, du, osum,
     acc, idxg, idxs, r0, r1, r2, cbuf, g0, g1, g2, s0, s1, s2, ss) = (
        refs[0:12], refs[12], refs[13], refs[14], refs[15], refs[16],
        refs[17], refs[18], refs[19:34], refs[34], refs[35], refs[36],
        refs[37], refs[38], refs[39], refs[40], refs[41], refs[42],
        refs[43], refs[44], refs[45], refs[46], refs[47])

    cid = lax.axis_index("c")
    sid = lax.axis_index("s")
    bufs = (r0, r1, r2)
    gsems = (g0, g1, g2)
    ssems = (s0, s1, s2)

    z16 = jnp.zeros((16,), _f32)
    one16 = jnp.where(lax.iota(jnp.int32, 16) == 0, 1.0, 0.0).astype(_f32)

    @pl.loop(0, K)
    def _(i):
        cbuf[i, pl.ds(0, 16)] = one16
        cbuf[i, pl.ds(16, 16)] = z16

    def one_pass(gather_tbl, seq_tbl, src_hbm, dst_hbm, out_hbm):
        @pl.loop(0, K)
        def _(i):
            r0[i, pl.ds(0, 16)] = z16
            r0[i, pl.ds(16, 16)] = z16
        base = sid * RPW
        for k in range(RPW // K):
            pltpu.sync_copy(r0, acc.at[pl.ds(base + k * K, K)])
        pltpu.sync_copy(r0.at[pl.ds(0, RPW % K)],
                        acc.at[pl.ds(base + (RPW // K) * K, RPW % K)])
        plsc.subcore_barrier()

        for h in range(BPW // HB):
            pltpu.sync_copy(dst_hbm.at[sid, pl.ds(h * HB, HB)], idxs)
            if gather_tbl is not None:
                pltpu.sync_copy(src_hbm.at[sid, pl.ds(h * HB, HB)], idxg)

            if gather_tbl is None and seq_tbl is None:
                @pl.loop(0, HB)
                def _(j):
                    pltpu.async_copy(cbuf, acc.at[idxs.at[j]], ss, add=True)

                @pl.loop(0, HB)
                def _(j):
                    pltpu.make_async_copy(cbuf, acc.at[idxs.at[j]], ss).wait()
            else:
                seq0 = sid * EPW + h * HB * K

                def g_desc(j, b):
                    if gather_tbl is not None:
                        return (gather_tbl.at[idxg.at[j]], bufs[b], gsems[b])
                    return (seq_tbl.at[pl.ds(seq0 + j * K, K)],
                            bufs[b], gsems[b])

                def s_desc(j, b):
                    return (bufs[b], acc.at[idxs.at[j]], ssems[b])

                pltpu.async_copy(*g_desc(0, 0))
                pltpu.async_copy(*g_desc(1, 1))

                @pl.loop(0, HB, step=3)
                def _(j0):
                    for b in range(3):
                        j = j0 + b

                        @pl.when(j < HB)
                        def _(j=j, b=b, g_desc=g_desc, s_desc=s_desc):
                            pltpu.make_async_copy(*g_desc(j, b)).wait()
                            pltpu.async_copy(*s_desc(j, b), add=True)

                            @pl.when(j >= 1)
                            def _(j=j, b=b, s_desc=s_desc):
                                pltpu.make_async_copy(
                                    *s_desc(j - 1, (b + 2) % 3)).wait()

                            @pl.when(j + 2 < HB)
                            def _(j=j, b=b, g_desc=g_desc):
                                pltpu.async_copy(*g_desc(j + 2, (b + 2) % 3))

                pltpu.make_async_copy(*s_desc(HB - 1, (HB - 1) % 3)).wait()

        plsc.subcore_barrier()
        pltpu.sync_copy(acc.at[pl.ds(sid * RPW, RPW)],
                        out_hbm.at[pl.ds(sid * RPW, RPW)])
        plsc.subcore_barrier()

    passes = []
    for g in range(NG):
        owner = 0 if g < 2 else 1
        passes.append((tbls[g], None, sp, dp, osum[g], owner))
        passes.append((tbls[NG + g], None, sc_, dc_, osum[NG + g], owner))
        passes.append((tbls[2 * NG + g], None, su, du, osum[2 * NG + g], owner))
    passes.append((None, ef32, None, dc_, osum[12], 0))
    passes.append((None, None, None, dp, osum[13], 1))
    passes.append((None, None, None, du, osum[14], 1))

    for gt, st, s_h, d_h, o_h, owner in passes:
        @pl.when(cid == owner)
        def _(gt=gt, st=st, s_h=s_h, d_h=d_h, o_h=o_h):
            one_pass(gt, st, s_h, d_h, o_h)


def _sc_segment_sums(tables, ef32, sp, dp, sc_, dc_, su, du):
    mesh = plsc.VectorSubcoreMesh(core_axis_name="c", subcore_axis_name="s")
    kern = pl.kernel(
        _sc_body,
        out_type=[jax.ShapeDtypeStruct((NPAD, G), _f32)] * 15,
        mesh=mesh,
        compiler_params=pltpu.CompilerParams(use_tc_tiling_on_sc=False),
        scratch_types=[
            pltpu.VMEM_SHARED((NPAD, G), _f32),
            pltpu.VMEM((HB, K), jnp.int32),
            pltpu.VMEM((HB, K), jnp.int32),
            pltpu.VMEM((K, G), _f32),
            pltpu.VMEM((K, G), _f32),
            pltpu.VMEM((K, G), _f32),
            pltpu.VMEM((K, G), _f32),
            pltpu.SemaphoreType.DMA,
            pltpu.SemaphoreType.DMA,
            pltpu.SemaphoreType.DMA,
            pltpu.SemaphoreType.DMA,
            pltpu.SemaphoreType.DMA,
            pltpu.SemaphoreType.DMA,
            pltpu.SemaphoreType.DMA,
        ],
    )
    return kern(*tables, ef32, sp, dp, sc_, dc_, su, du)


def _fin_body(wep, bep, *refs):
    sums = refs[0:12]
    efs, cp, cu = refs[12], refs[13], refs[14]
    pub, com, ucu = refs[15], refs[16], refs[17]
    cnt_c = efs[...][:, CONV:CONV + 1]
    invp = 1.0 / jnp.maximum(cp[...][:, 0:1], 1.0)
    invc = 1.0 / jnp.maximum(cnt_c, 1.0)
    invu = 1.0 / jnp.maximum(cu[...][:, 0:1], 1.0)
    pub[...] = jnp.concatenate([sums[g][...] for g in range(NG)], axis=1) * invp
    nonzero = jnp.where(cnt_c >= 1.0, 0.3, 0.0)
    base = (jnp.dot(efs[...][:, 0:CONV] * invc, wep[...],
                    preferred_element_type=_f32) + bep[...]) * nonzero
    com[...] = jnp.concatenate([sums[NG + g][...] for g in range(NG)],
                               axis=1) * invc + base
    ucu[...] = jnp.concatenate([sums[2 * NG + g][...] for g in range(NG)],
                               axis=1) * invu


def _finalize(W_ep, b_ep, sums):
    blk = 1000
    grid = (N_NODE // blk,)
    return pl.pallas_call(
        _fin_body,
        grid=grid,
        in_specs=[pl.BlockSpec((CONV, D), lambda i: (0, 0)),
                  pl.BlockSpec((1, D), lambda i: (0, 0))] +
                 [pl.BlockSpec((blk, G), lambda i: (i, 0))] * 15,
        out_specs=[pl.BlockSpec((blk, D), lambda i: (i, 0))] * 3,
        out_shape=[jax.ShapeDtypeStruct((N_NODE, D), _f32)] * 3,
    )(W_ep, b_ep.reshape(1, D), *sums)


@jax.jit
def kernel(h_user, h_post, user_context, edge_feat_comment, W_pub, b_pub,
           W_com, b_com, W_conv, b_conv, ln_g, ln_b, W_ep, b_ep,
           edge_index_publish, edge_index_comment, edge_index_ucu):
    tables = _node_tables(h_user, user_context, W_pub, b_pub, W_com, b_com,
                          W_conv[:D], W_conv[D:], b_conv, ln_g, ln_b)
    ef32 = jnp.concatenate(
        [edge_feat_comment, jnp.ones((E, 1), _f32),
         jnp.zeros((E, G - CONV - 1), _f32)], axis=1)
    ef32 = jnp.concatenate([ef32, jnp.zeros((EPAD - E, G), _f32)], axis=0)
    i32 = jnp.int32
    zpad = jnp.zeros((EPAD - E,), i32)
    tpad = TRASH + jnp.arange(EPAD - E, dtype=i32) % (NPAD - TRASH)

    def _src(row):
        return jnp.concatenate([row.astype(i32), zpad]).reshape(NS, BPW, K)

    def _dst(row):
        return jnp.concatenate([row.astype(i32), tpad]).reshape(NS, BPW, K)

    sp, dp = _src(edge_index_publish[0]), _dst(edge_index_publish[1])
    sc_, dc_ = _src(edge_index_comment[0]), _dst(edge_index_comment[1])
    su, du = _src(edge_index_ucu[0]), _dst(edge_index_ucu[1])
    sums = _sc_segment_sums(tables, ef32, sp, dp, sc_, dc_, su, du)
    pub, com, ucu = _finalize(W_ep, b_ep, sums)
    return (pub, com, ucu)


# speedup vs baseline: 1.1798x; 1.1760x over previous
import functools

import jax
import jax.numpy as jnp
from jax import lax
from jax.experimental import pallas as pl
from jax.experimental.pallas import tpu as pltpu
from jax.experimental.pallas import tpu_sc as plsc

N_NODE = 50000
E = 160000
D = 128
G = 32
NG = D // G
CONV = 16

NS = 16
K = 80
BPW = 125
HB = BPW
EPW = K * BPW
EPAD = NS * EPW
NPAD = 50048
RPW = NPAD // NS
TRASH = N_NODE

_f32 = jnp.float32


def _tables_body(hu, uc, wp, bp, wc, bc, w1, w2, bv, lg, lb, *outs):
    x = hu[...]
    tp = jnp.dot(x, wp[...], preferred_element_type=_f32) + bp[...]
    tc = (jnp.dot(x, wc[...], preferred_element_type=_f32) + bc[...]) * 0.7
    z = (jnp.dot(x, w1[...], preferred_element_type=_f32)
         + jnp.dot(uc[...], w2[...], preferred_element_type=_f32) + bv[...])
    mu = jnp.mean(z, axis=-1, keepdims=True)
    var = jnp.mean((z - mu) ** 2, axis=-1, keepdims=True)
    nm = jnp.maximum((z - mu) / jnp.sqrt(var + 1e-5) * lg[...] + lb[...], 0.0)
    for g in range(NG):
        outs[g][...] = tp[:, G * g:G * g + G]
        outs[NG + g][...] = tc[:, G * g:G * g + G]
        outs[2 * NG + g][...] = nm[:, G * g:G * g + G]


def _node_tables(h_user, user_context, W_pub, b_pub, W_com, b_com,
                 W1, W2, b_conv, ln_g, ln_b):
    blk = 1000
    grid = (N_NODE // blk,)
    full = lambda r, c: pl.BlockSpec((r, c), lambda i: (0, 0))
    return pl.pallas_call(
        _tables_body,
        grid=grid,
        in_specs=[
            pl.BlockSpec((blk, D), lambda i: (i, 0)),
            pl.BlockSpec((blk, CONV), lambda i: (i, 0)),
            full(D, D), full(1, D), full(D, D), full(1, D),
            full(D, D), full(CONV, D), full(1, D), full(1, D), full(1, D),
        ],
        out_specs=[pl.BlockSpec((blk, G), lambda i: (i, 0))] * (3 * NG),
        out_shape=[jax.ShapeDtypeStruct((N_NODE, G), _f32)] * (3 * NG),
    )(h_user, user_context, W_pub, b_pub.reshape(1, D), W_com,
      b_com.reshape(1, D), W1, W2, b_conv.reshape(1, D),
      ln_g.reshape(1, D), ln_b.reshape(1, D))


def _sc_body(*refs):
    (tbls, ef32, sp, dp, sc_, dc_, su, du, osum,
     acc, idxg, idxs, r0, r1, r2, cbuf, g0, g1, g2, s0, s1, s2, ss) = (
        refs[0:12], refs[12], refs[13], refs[14], refs[15], refs[16],
        refs[17], refs[18], refs[19:34], refs[34], refs[35], refs[36],
        refs[37], refs[38], refs[39], refs[40], refs[41], refs[42],
        refs[43], refs[44], refs[45], refs[46], refs[47])

    cid = lax.axis_index("c")
    sid = lax.axis_index("s")
    bufs = (r0, r1, r2)
    gsems = (g0, g1, g2)
    ssems = (s0, s1, s2)

    z16 = jnp.zeros((16,), _f32)
    one16 = jnp.where(lax.iota(jnp.int32, 16) == 0, 1.0, 0.0).astype(_f32)

    @pl.loop(0, K)
    def _(i):
        cbuf[i, pl.ds(0, 16)] = one16
        cbuf[i, pl.ds(16, 16)] = z16

    def one_pass(gather_tbl, seq_tbl, src_hbm, dst_hbm, out_hbm):
        @pl.loop(0, K)
        def _(i):
            r0[i, pl.ds(0, 16)] = z16
            r0[i, pl.ds(16, 16)] = z16
        base = sid * RPW
        for k in range(RPW // K):
            pltpu.sync_copy(r0, acc.at[pl.ds(base + k * K, K)])
        pltpu.sync_copy(r0.at[pl.ds(0, RPW % K)],
                        acc.at[pl.ds(base + (RPW // K) * K, RPW % K)])
        plsc.subcore_barrier()

        for h in range(BPW // HB):
            pltpu.sync_copy(dst_hbm.at[sid, pl.ds(h * HB, HB)], idxs)
            if gather_tbl is not None:
                pltpu.sync_copy(src_hbm.at[sid, pl.ds(h * HB, HB)], idxg)

            if gather_tbl is None and seq_tbl is None:
                @pl.loop(0, HB)
                def _(j):
                    pltpu.async_copy(cbuf, acc.at[idxs.at[j]], ss, add=True)

                @pl.loop(0, HB)
                def _(j):
                    pltpu.make_async_copy(cbuf, acc.at[idxs.at[j]], ss).wait()
            else:
                seq0 = sid * EPW + h * HB * K

                def g_desc(j, b):
                    if gather_tbl is not None:
                        return (gather_tbl.at[idxg.at[j]], bufs[b], gsems[b])
                    return (seq_tbl.at[pl.ds(seq0 + j * K, K)],
                            bufs[b], gsems[b])

                def s_desc(j, b):
                    return (bufs[b], acc.at[idxs.at[j]], ssems[b])

                pltpu.async_copy(*g_desc(0, 0))
                pltpu.async_copy(*g_desc(1, 1))

                @pl.loop(0, HB, step=3)
                def _(j0):
                    for b in range(3):
                        j = j0 + b

                        @pl.when(j < HB)
                        def _(j=j, b=b, g_desc=g_desc, s_desc=s_desc):
                            pltpu.make_async_copy(*g_desc(j, b)).wait()
                            pltpu.async_copy(*s_desc(j, b), add=True)

                            @pl.when(j >= 1)
                            def _(j=j, b=b, s_desc=s_desc):
                                pltpu.make_async_copy(
                                    *s_desc(j - 1, (b + 2) % 3)).wait()

                            @pl.when(j + 2 < HB)
                            def _(j=j, b=b, g_desc=g_desc):
                                pltpu.async_copy(*g_desc(j + 2, (b + 2) % 3))

                pltpu.make_async_copy(*s_desc(HB - 1, (HB - 1) % 3)).wait()

        plsc.subcore_barrier()
        pltpu.sync_copy(acc.at[pl.ds(sid * RPW, RPW)],
                        out_hbm.at[pl.ds(sid * RPW, RPW)])
        plsc.subcore_barrier()

    passes = []
    for g in range(NG):
        owner = 0 if g < 2 else 1
        passes.append((tbls[g], None, sp, dp, osum[g], owner))
        passes.append((tbls[NG + g], None, sc_, dc_, osum[NG + g], owner))
        passes.append((tbls[2 * NG + g], None, su, du, osum[2 * NG + g], owner))
    passes.append((None, ef32, None, dc_, osum[12], 0))
    passes.append((None, None, None, dp, osum[13], 1))
    passes.append((None, None, None, du, osum[14], 1))

    for gt, st, s_h, d_h, o_h, owner in passes:
        @pl.when(cid == owner)
        def _(gt=gt, st=st, s_h=s_h, d_h=d_h, o_h=o_h):
            one_pass(gt, st, s_h, d_h, o_h)


def _sc_segment_sums(tables, ef32, sp, dp, sc_, dc_, su, du):
    mesh = plsc.VectorSubcoreMesh(core_axis_name="c", subcore_axis_name="s")
    kern = pl.kernel(
        _sc_body,
        out_type=[jax.ShapeDtypeStruct((NPAD, G), _f32)] * 15,
        mesh=mesh,
        compiler_params=pltpu.CompilerParams(use_tc_tiling_on_sc=False),
        scratch_types=[
            pltpu.VMEM_SHARED((NPAD, G), _f32),
            pltpu.VMEM((HB, K), jnp.int32),
            pltpu.VMEM((HB, K), jnp.int32),
            pltpu.VMEM((K, G), _f32),
            pltpu.VMEM((K, G), _f32),
            pltpu.VMEM((K, G), _f32),
            pltpu.VMEM((K, G), _f32),
            pltpu.SemaphoreType.DMA,
            pltpu.SemaphoreType.DMA,
            pltpu.SemaphoreType.DMA,
            pltpu.SemaphoreType.DMA,
            pltpu.SemaphoreType.DMA,
            pltpu.SemaphoreType.DMA,
            pltpu.SemaphoreType.DMA,
        ],
    )
    return kern(*tables, ef32, sp, dp, sc_, dc_, su, du)


def _fin_body(wep, bep, *refs):
    sums = refs[0:12]
    efs, cp, cu = refs[12], refs[13], refs[14]
    pub, com, ucu = refs[15], refs[16], refs[17]
    cnt_c = efs[...][:, CONV:CONV + 1]
    invp = 1.0 / jnp.maximum(cp[...][:, 0:1], 1.0)
    invc = 1.0 / jnp.maximum(cnt_c, 1.0)
    invu = 1.0 / jnp.maximum(cu[...][:, 0:1], 1.0)
    pub[...] = jnp.concatenate([sums[g][...] for g in range(NG)], axis=1) * invp
    nonzero = jnp.where(cnt_c >= 1.0, 0.3, 0.0)
    base = (jnp.dot(efs[...][:, 0:CONV] * invc, wep[...],
                    preferred_element_type=_f32) + bep[...]) * nonzero
    com[...] = jnp.concatenate([sums[NG + g][...] for g in range(NG)],
                               axis=1) * invc + base
    ucu[...] = jnp.concatenate([sums[2 * NG + g][...] for g in range(NG)],
                               axis=1) * invu


def _finalize(W_ep, b_ep, sums):
    blk = 1000
    grid = (N_NODE // blk,)
    return pl.pallas_call(
        _fin_body,
        grid=grid,
        in_specs=[pl.BlockSpec((CONV, D), lambda i: (0, 0)),
                  pl.BlockSpec((1, D), lambda i: (0, 0))] +
                 [pl.BlockSpec((blk, G), lambda i: (i, 0))] * 15,
        out_specs=[pl.BlockSpec((blk, D), lambda i: (i, 0))] * 3,
        out_shape=[jax.ShapeDtypeStruct((N_NODE, D), _f32)] * 3,
    )(W_ep, b_ep.reshape(1, D), *sums)


@jax.jit
def kernel(h_user, h_post, user_context, edge_feat_comment, W_pub, b_pub,
           W_com, b_com, W_conv, b_conv, ln_g, ln_b, W_ep, b_ep,
           edge_index_publish, edge_index_comment, edge_index_ucu):
    tables = _node_tables(h_user, user_context, W_pub, b_pub, W_com, b_com,
                          W_conv[:D], W_conv[D:], b_conv, ln_g, ln_b)
    ef32 = jnp.concatenate(
        [edge_feat_comment, jnp.ones((E, 1), _f32),
         jnp.zeros((E, G - CONV - 1), _f32)], axis=1)
    ef32 = jnp.concatenate([ef32, jnp.zeros((EPAD - E, G), _f32)], axis=0)
    i32 = jnp.int32
    zpad = jnp.zeros((EPAD - E,), i32)
    tpad = TRASH + jnp.arange(EPAD - E, dtype=i32) % (NPAD - TRASH)

    def _src(row):
        return jnp.concatenate([row.astype(i32), zpad]).reshape(NS, BPW, K)

    def _dst(row):
        return jnp.concatenate([row.astype(i32), tpad]).reshape(NS, BPW, K)

    sp, dp = _src(edge_index_publish[0]), _dst(edge_index_publish[1])
    sc_, dc_ = _src(edge_index_comment[0]), _dst(edge_index_comment[1])
    su, du = _src(edge_index_ucu[0]), _dst(edge_index_ucu[1])
    sums = _sc_segment_sums(tables, ef32, sp, dp, sc_, dc_, su, du)
    pub, com, ucu = _finalize(W_ep, b_ep, sums)
    return (pub, com, ucu)


# speedup vs baseline: 1.1798x; 1.0000x over previous
import functools

import jax
import jax.numpy as jnp
from jax import lax
from jax.experimental import pallas as pl
from jax.experimental.pallas import tpu as pltpu
from jax.experimental.pallas import tpu_sc as plsc

N_NODE = 50000
E = 160000
D = 128
G = 32
NG = D // G
CONV = 16

NS = 16
K = 80
BPW = 125
HB = BPW
EPW = K * BPW
EPAD = NS * EPW
NPAD = 50048
RPW = NPAD // NS
TRASH = N_NODE

_f32 = jnp.float32


def _tables_body(hu, uc, wp, bp, wc, bc, w1, w2, bv, lg, lb, *outs):
    x = hu[...]
    tp = jnp.dot(x, wp[...], preferred_element_type=_f32) + bp[...]
    tc = (jnp.dot(x, wc[...], preferred_element_type=_f32) + bc[...]) * 0.7
    z = (jnp.dot(x, w1[...], preferred_element_type=_f32)
         + jnp.dot(uc[...], w2[...], preferred_element_type=_f32) + bv[...])
    mu = jnp.mean(z, axis=-1, keepdims=True)
    var = jnp.mean((z - mu) ** 2, axis=-1, keepdims=True)
    nm = jnp.maximum((z - mu) / jnp.sqrt(var + 1e-5) * lg[...] + lb[...], 0.0)
    for g in range(NG):
        outs[g][...] = tp[:, G * g:G * g + G]
        outs[NG + g][...] = tc[:, G * g:G * g + G]
        outs[2 * NG + g][...] = nm[:, G * g:G * g + G]


def _node_tables(h_user, user_context, W_pub, b_pub, W_com, b_com,
                 W1, W2, b_conv, ln_g, ln_b):
    blk = 1000
    grid = (N_NODE // blk,)
    full = lambda r, c: pl.BlockSpec((r, c), lambda i: (0, 0))
    return pl.pallas_call(
        _tables_body,
        grid=grid,
        in_specs=[
            pl.BlockSpec((blk, D), lambda i: (i, 0)),
            pl.BlockSpec((blk, CONV), lambda i: (i, 0)),
            full(D, D), full(1, D), full(D, D), full(1, D),
            full(D, D), full(CONV, D), full(1, D), full(1, D), full(1, D),
        ],
        out_specs=[pl.BlockSpec((blk, G), lambda i: (i, 0))] * (3 * NG),
        out_shape=[jax.ShapeDtypeStruct((N_NODE, G), _f32)] * (3 * NG),
    )(h_user, user_context, W_pub, b_pub.reshape(1, D), W_com,
      b_com.reshape(1, D), W1, W2, b_conv.reshape(1, D),
      ln_g.reshape(1, D), ln_b.reshape(1, D))


def _sc_body(*refs):
    (tbls, ef32, sp, dp, sc_, dc_, su, du, osum,
     acc, idxg, idxs, r0, r1, r2, cbuf, g0, g1, g2, s0, s1, s2, ss) = (
        refs[0:12], refs[12], refs[13], refs[14], refs[15], refs[16],
        refs[17], refs[18], refs[19:34], refs[34], refs[35], refs[36],
        refs[37], refs[38], refs[39], refs[40], refs[41], refs[42],
        refs[43], refs[44], refs[45], refs[46], refs[47])

    cid = lax.axis_index("c")
    sid = lax.axis_index("s")
    bufs = (r0, r1, r2)
    gsems = (g0, g1, g2)
    ssems = (s0, s1, s2)

    z16 = jnp.zeros((16,), _f32)
    one16 = jnp.where(lax.iota(jnp.int32, 16) == 0, 1.0, 0.0).astype(_f32)

    @pl.loop(0, K)
    def _(i):
        cbuf[i, pl.ds(0, 16)] = one16
        cbuf[i, pl.ds(16, 16)] = z16

    def one_pass(gather_tbl, seq_tbl, src_hbm, dst_hbm, out_hbm):
        @pl.loop(0, K)
        def _(i):
            r0[i, pl.ds(0, 16)] = z16
            r0[i, pl.ds(16, 16)] = z16
        base = sid * RPW
        for k in range(RPW // K):
            pltpu.sync_copy(r0, acc.at[pl.ds(base + k * K, K)])
        pltpu.sync_copy(r0.at[pl.ds(0, RPW % K)],
                        acc.at[pl.ds(base + (RPW // K) * K, RPW % K)])
        plsc.subcore_barrier()

        for h in range(BPW // HB):
            pltpu.sync_copy(dst_hbm.at[sid, pl.ds(h * HB, HB)], idxs)
            if gather_tbl is not None:
                pltpu.sync_copy(src_hbm.at[sid, pl.ds(h * HB, HB)], idxg)

            if gather_tbl is None and seq_tbl is None:
                @pl.loop(0, HB)
                def _(j):
                    pltpu.async_copy(cbuf, acc.at[idxs.at[j]], ss, add=True)

                @pl.loop(0, HB)
                def _(j):
                    pltpu.make_async_copy(cbuf, acc.at[idxs.at[j]], ss).wait()
            else:
                seq0 = sid * EPW + h * HB * K

                def g_desc(j, b):
                    if gather_tbl is not None:
                        return (gather_tbl.at[idxg.at[j]], bufs[b], gsems[b])
                    return (seq_tbl.at[pl.ds(seq0 + j * K, K)],
                            bufs[b], gsems[b])

                def s_desc(j, b):
                    return (bufs[b], acc.at[idxs.at[j]], ssems[b])

                pltpu.async_copy(*g_desc(0, 0))
                pltpu.async_copy(*g_desc(1, 1))

                @pl.loop(0, HB, step=3)
                def _(j0):
                    for b in range(3):
                        j = j0 + b

                        @pl.when(j < HB)
                        def _(j=j, b=b, g_desc=g_desc, s_desc=s_desc):
                            pltpu.make_async_copy(*g_desc(j, b)).wait()
                            pltpu.async_copy(*s_desc(j, b), add=True)

                            @pl.when(j >= 1)
                            def _(j=j, b=b, s_desc=s_desc):
                                pltpu.make_async_copy(
                                    *s_desc(j - 1, (b + 2) % 3)).wait()

                            @pl.when(j + 2 < HB)
                            def _(j=j, b=b, g_desc=g_desc):
                                pltpu.async_copy(*g_desc(j + 2, (b + 2) % 3))

                pltpu.make_async_copy(*s_desc(HB - 1, (HB - 1) % 3)).wait()

        plsc.subcore_barrier()
        pltpu.sync_copy(acc.at[pl.ds(sid * RPW, RPW)],
                        out_hbm.at[pl.ds(sid * RPW, RPW)])
        plsc.subcore_barrier()

    passes = []
    for g in range(NG):
        owner = 0 if g < 2 else 1
        passes.append((tbls[g], None, sp, dp, osum[g], owner))
        passes.append((tbls[NG + g], None, sc_, dc_, osum[NG + g], owner))
        passes.append((tbls[2 * NG + g], None, su, du, osum[2 * NG + g], owner))
    passes.append((None, ef32, None, dc_, osum[12], 0))
    passes.append((None, None, None, dp, osum[13], 1))
    passes.append((None, None, None, du, osum[14], 1))

    for p, (gt, st, s_h, d_h, o_h, owner) in enumerate(passes):
        kind = "gather" if gt is not None else ("seq" if st is not None else "cnt")
        with jax.named_scope(f"pass{p:02d}_{kind}_c{owner}"):
            @pl.when(cid == owner)
            def _(gt=gt, st=st, s_h=s_h, d_h=d_h, o_h=o_h):
                one_pass(gt, st, s_h, d_h, o_h)


def _sc_segment_sums(tables, ef32, sp, dp, sc_, dc_, su, du):
    mesh = plsc.VectorSubcoreMesh(core_axis_name="c", subcore_axis_name="s")
    kern = pl.kernel(
        _sc_body,
        out_type=[jax.ShapeDtypeStruct((NPAD, G), _f32)] * 15,
        mesh=mesh,
        compiler_params=pltpu.CompilerParams(use_tc_tiling_on_sc=False),
        scratch_types=[
            pltpu.VMEM_SHARED((NPAD, G), _f32),
            pltpu.VMEM((HB, K), jnp.int32),
            pltpu.VMEM((HB, K), jnp.int32),
            pltpu.VMEM((K, G), _f32),
            pltpu.VMEM((K, G), _f32),
            pltpu.VMEM((K, G), _f32),
            pltpu.VMEM((K, G), _f32),
            pltpu.SemaphoreType.DMA,
            pltpu.SemaphoreType.DMA,
            pltpu.SemaphoreType.DMA,
            pltpu.SemaphoreType.DMA,
            pltpu.SemaphoreType.DMA,
            pltpu.SemaphoreType.DMA,
            pltpu.SemaphoreType.DMA,
        ],
    )
    return kern(*tables, ef32, sp, dp, sc_, dc_, su, du)


def _fin_body(wep, bep, *refs):
    sums = refs[0:12]
    efs, cp, cu = refs[12], refs[13], refs[14]
    pub, com, ucu = refs[15], refs[16], refs[17]
    cnt_c = efs[...][:, CONV:CONV + 1]
    invp = 1.0 / jnp.maximum(cp[...][:, 0:1], 1.0)
    invc = 1.0 / jnp.maximum(cnt_c, 1.0)
    invu = 1.0 / jnp.maximum(cu[...][:, 0:1], 1.0)
    pub[...] = jnp.concatenate([sums[g][...] for g in range(NG)], axis=1) * invp
    nonzero = jnp.where(cnt_c >= 1.0, 0.3, 0.0)
    base = (jnp.dot(efs[...][:, 0:CONV] * invc, wep[...],
                    preferred_element_type=_f32) + bep[...]) * nonzero
    com[...] = jnp.concatenate([sums[NG + g][...] for g in range(NG)],
                               axis=1) * invc + base
    ucu[...] = jnp.concatenate([sums[2 * NG + g][...] for g in range(NG)],
                               axis=1) * invu


def _finalize(W_ep, b_ep, sums):
    blk = 1000
    grid = (N_NODE // blk,)
    return pl.pallas_call(
        _fin_body,
        grid=grid,
        in_specs=[pl.BlockSpec((CONV, D), lambda i: (0, 0)),
                  pl.BlockSpec((1, D), lambda i: (0, 0))] +
                 [pl.BlockSpec((blk, G), lambda i: (i, 0))] * 15,
        out_specs=[pl.BlockSpec((blk, D), lambda i: (i, 0))] * 3,
        out_shape=[jax.ShapeDtypeStruct((N_NODE, D), _f32)] * 3,
    )(W_ep, b_ep.reshape(1, D), *sums)


@jax.jit
def kernel(h_user, h_post, user_context, edge_feat_comment, W_pub, b_pub,
           W_com, b_com, W_conv, b_conv, ln_g, ln_b, W_ep, b_ep,
           edge_index_publish, edge_index_comment, edge_index_ucu):
    tables = _node_tables(h_user, user_context, W_pub, b_pub, W_com, b_com,
                          W_conv[:D], W_conv[D:], b_conv, ln_g, ln_b)
    ef32 = jnp.concatenate(
        [edge_feat_comment, jnp.ones((E, 1), _f32),
         jnp.zeros((E, G - CONV - 1), _f32)], axis=1)
    ef32 = jnp.concatenate([ef32, jnp.zeros((EPAD - E, G), _f32)], axis=0)
    i32 = jnp.int32
    zpad = jnp.zeros((EPAD - E,), i32)
    tpad = TRASH + jnp.arange(EPAD - E, dtype=i32) % (NPAD - TRASH)

    def _src(row):
        return jnp.concatenate([row.astype(i32), zpad]).reshape(NS, BPW, K)

    def _dst(row):
        return jnp.concatenate([row.astype(i32), tpad]).reshape(NS, BPW, K)

    sp, dp = _src(edge_index_publish[0]), _dst(edge_index_publish[1])
    sc_, dc_ = _src(edge_index_comment[0]), _dst(edge_index_comment[1])
    su, du = _src(edge_index_ucu[0]), _dst(edge_index_ucu[1])
    sums = _sc_segment_sums(tables, ef32, sp, dp, sc_, dc_, su, du)
    pub, com, ucu = _finalize(W_ep, b_ep, sums)
    return (pub, com, ucu)


# speedup vs baseline: 2.0580x; 1.7443x over previous
import functools

import jax
import jax.numpy as jnp
from jax import lax
from jax.experimental import pallas as pl
from jax.experimental.pallas import tpu as pltpu
from jax.experimental.pallas import tpu_sc as plsc

N_NODE = 50000
E = 160000
D = 128
G = 32
NG = D // G
CONV = 16

NS = 16
K = 80
BPW = 125
HB = BPW
EPW = K * BPW
EPAD = NS * EPW
NPAD = 50048
RPW = NPAD // NS
TRASH = N_NODE

_f32 = jnp.float32


def _tables_body(hu, uc, wp, bp, wc, bc, w1, w2, bv, lg, lb, *outs):
    x = hu[...]
    tp = jnp.dot(x, wp[...], preferred_element_type=_f32) + bp[...]
    tc = (jnp.dot(x, wc[...], preferred_element_type=_f32) + bc[...]) * 0.7
    z = (jnp.dot(x, w1[...], preferred_element_type=_f32)
         + jnp.dot(uc[...], w2[...], preferred_element_type=_f32) + bv[...])
    mu = jnp.mean(z, axis=-1, keepdims=True)
    var = jnp.mean((z - mu) ** 2, axis=-1, keepdims=True)
    nm = jnp.maximum((z - mu) / jnp.sqrt(var + 1e-5) * lg[...] + lb[...], 0.0)
    outs[0][...] = tp
    outs[1][...] = tc
    outs[2][...] = nm


def _node_tables(h_user, user_context, W_pub, b_pub, W_com, b_com,
                 W1, W2, b_conv, ln_g, ln_b):
    blk = 2048
    grid = ((N_NODE + blk - 1) // blk,)
    full = lambda r, c: pl.BlockSpec((r, c), lambda i: (0, 0))
    return pl.pallas_call(
        _tables_body,
        grid=grid,
        in_specs=[
            pl.BlockSpec((blk, D), lambda i: (i, 0)),
            pl.BlockSpec((blk, CONV), lambda i: (i, 0)),
            full(D, D), full(1, D), full(D, D), full(1, D),
            full(D, D), full(CONV, D), full(1, D), full(1, D), full(1, D),
        ],
        out_specs=[pl.BlockSpec((blk, D), lambda i: (i, 0))] * 3,
        out_shape=[jax.ShapeDtypeStruct((N_NODE, D), _f32)] * 3,
    )(h_user, user_context, W_pub, b_pub.reshape(1, D), W_com,
      b_com.reshape(1, D), W1, W2, b_conv.reshape(1, D),
      ln_g.reshape(1, D), ln_b.reshape(1, D))


def _sc_body(*refs):
    (tbls, ef32, eip, eic, eiu, opub, ocom, oucu, aux,
     acc, idxg, idxs, r0, r1, r2, cbuf, g0, g1, g2, s0, s1, s2, ss) = (
        refs[0:3], refs[3], refs[4], refs[5], refs[6], refs[7], refs[8],
        refs[9], refs[10], refs[11], refs[12], refs[13], refs[14], refs[15],
        refs[16], refs[17], refs[18], refs[19], refs[20], refs[21], refs[22],
        refs[23], refs[24])

    cid = lax.axis_index("c")
    sid = lax.axis_index("s")
    bufs = (r0, r1, r2)
    gsems = (g0, g1, g2)
    ssems = (s0, s1, s2)

    z16 = jnp.zeros((16,), _f32)
    one16 = jnp.where(lax.iota(jnp.int32, 16) == 0, 1.0, 0.0).astype(_f32)

    @pl.loop(0, K)
    def _(i):
        cbuf[i, pl.ds(0, 16)] = one16
        cbuf[i, pl.ds(16, 16)] = z16

    def one_pass(gather_tbl, grp, seq_tbl, eidx, out_hbm, col):
        @pl.loop(0, K)
        def _(i):
            r0[i, pl.ds(0, 16)] = z16
            r0[i, pl.ds(16, 16)] = z16
        base = sid * RPW
        for k in range(RPW // K):
            pltpu.sync_copy(r0, acc.at[pl.ds(base + k * K, K)])
        pltpu.sync_copy(r0.at[pl.ds(0, RPW % K)],
                        acc.at[pl.ds(base + (RPW // K) * K, RPW % K)])
        plsc.subcore_barrier()

        pltpu.sync_copy(eidx.at[1, pl.ds(sid * EPW, EPW)], idxs)
        if gather_tbl is not None:
            pltpu.sync_copy(eidx.at[0, pl.ds(sid * EPW, EPW)], idxg)

            @pl.loop(0, EPW // 16)
            def _(i):
                v = idxg[pl.ds(i * 16, 16)]
                idxg[pl.ds(i * 16, 16)] = v * 4 + grp

        if gather_tbl is None and seq_tbl is None:
            @pl.loop(0, BPW)
            def _(j):
                pltpu.async_copy(cbuf, acc.at[idxs.at[pl.ds(j * K, K)]],
                                 ss, add=True)

            @pl.loop(0, BPW)
            def _(j):
                pltpu.make_async_copy(cbuf, acc.at[idxs.at[pl.ds(j * K, K)]],
                                      ss).wait()
        else:
            def g_desc(j, b):
                if gather_tbl is not None:
                    return (gather_tbl.at[idxg.at[pl.ds(j * K, K)]],
                            bufs[b], gsems[b])
                return (seq_tbl.at[pl.ds(sid * EPW + j * K, K)],
                        bufs[b], gsems[b])

            def s_desc(j, b):
                return (bufs[b], acc.at[idxs.at[pl.ds(j * K, K)]], ssems[b])

            pltpu.async_copy(*g_desc(0, 0))
            pltpu.async_copy(*g_desc(1, 1))

            @pl.loop(0, BPW, step=3)
            def _(j0):
                for b in range(3):
                    j = j0 + b

                    @pl.when(j < BPW)
                    def _(j=j, b=b):
                        pltpu.make_async_copy(*g_desc(j, b)).wait()
                        pltpu.async_copy(*s_desc(j, b), add=True)

                        @pl.when(j >= 1)
                        def _(j=j, b=b):
                            pltpu.make_async_copy(
                                *s_desc(j - 1, (b + 2) % 3)).wait()

                        @pl.when(j + 2 < BPW)
                        def _(j=j, b=b):
                            pltpu.async_copy(*g_desc(j + 2, (b + 2) % 3))

            pltpu.make_async_copy(*s_desc(BPW - 1, (BPW - 1) % 3)).wait()

        plsc.subcore_barrier()
        pltpu.sync_copy(acc.at[pl.ds(sid * RPW, RPW)],
                        out_hbm.at[pl.ds(sid * RPW, RPW), pl.ds(col, G)])
        plsc.subcore_barrier()

    passes = []
    for g in range(NG):
        owner = 0 if g < 2 else 1
        passes.append((tbls[0], g, None, eip, opub, G * g, owner))
        passes.append((tbls[1], g, None, eic, ocom, G * g, owner))
        passes.append((tbls[2], g, None, eiu, oucu, G * g, owner))
    passes.append((None, 0, ef32, eic, aux, 0, 0))
    passes.append((None, 0, None, eip, aux, 32, 1))
    passes.append((None, 0, None, eiu, aux, 64, 1))

    for p, (gt, grp, st, e_h, o_h, col, owner) in enumerate(passes):
        kind = "gather" if gt is not None else ("seq" if st is not None else "cnt")
        with jax.named_scope(f"pass{p:02d}_{kind}_c{owner}"):
            @pl.when(cid == owner)
            def _(gt=gt, grp=grp, st=st, e_h=e_h, o_h=o_h, col=col):
                one_pass(gt, grp, st, e_h, o_h, col)


def _sc_segment_sums(tables, ef32, eip, eic, eiu):
    mesh = plsc.VectorSubcoreMesh(core_axis_name="c", subcore_axis_name="s")
    kern = pl.kernel(
        _sc_body,
        out_type=[jax.ShapeDtypeStruct((NPAD, D), _f32)] * 4,
        mesh=mesh,
        compiler_params=pltpu.CompilerParams(use_tc_tiling_on_sc=False),
        scratch_types=[
            pltpu.VMEM_SHARED((NPAD, G), _f32),
            pltpu.VMEM((EPW,), jnp.int32),
            pltpu.VMEM((EPW,), jnp.int32),
            pltpu.VMEM((K, G), _f32),
            pltpu.VMEM((K, G), _f32),
            pltpu.VMEM((K, G), _f32),
            pltpu.VMEM((K, G), _f32),
            pltpu.SemaphoreType.DMA,
            pltpu.SemaphoreType.DMA,
            pltpu.SemaphoreType.DMA,
            pltpu.SemaphoreType.DMA,
            pltpu.SemaphoreType.DMA,
            pltpu.SemaphoreType.DMA,
            pltpu.SemaphoreType.DMA,
        ],
    )
    return kern(*tables, ef32, eip, eic, eiu)


def _fin_body(wep, bep, spub, scom, sucu, aux, pub, com, ucu):
    a = aux[...]
    cnt_c = a[:, CONV:CONV + 1]
    invp = 1.0 / jnp.maximum(a[:, 32:33], 1.0)
    invc = 1.0 / jnp.maximum(cnt_c, 1.0)
    invu = 1.0 / jnp.maximum(a[:, 64:65], 1.0)
    pub[...] = spub[...] * invp
    nonzero = jnp.where(cnt_c >= 1.0, 0.3, 0.0)
    base = (jnp.dot(a[:, 0:CONV] * invc, wep[...],
                    preferred_element_type=_f32) + bep[...]) * nonzero
    com[...] = scom[...] * invc + base
    ucu[...] = sucu[...] * invu


def _finalize(W_ep, b_ep, sums):
    blk = 1024
    grid = ((N_NODE + blk - 1) // blk,)
    return pl.pallas_call(
        _fin_body,
        grid=grid,
        in_specs=[pl.BlockSpec((CONV, D), lambda i: (0, 0)),
                  pl.BlockSpec((1, D), lambda i: (0, 0))] +
                 [pl.BlockSpec((blk, D), lambda i: (i, 0))] * 4,
        out_specs=[pl.BlockSpec((blk, D), lambda i: (i, 0))] * 3,
        out_shape=[jax.ShapeDtypeStruct((N_NODE, D), _f32)] * 3,
    )(W_ep, b_ep.reshape(1, D), *sums)


@jax.jit
def kernel(h_user, h_post, user_context, edge_feat_comment, W_pub, b_pub,
           W_com, b_com, W_conv, b_conv, ln_g, ln_b, W_ep, b_ep,
           edge_index_publish, edge_index_comment, edge_index_ucu):
    tables128 = _node_tables(h_user, user_context, W_pub, b_pub, W_com, b_com,
                             W_conv[:D], W_conv[D:], b_conv, ln_g, ln_b)
    tables = [t.reshape(N_NODE * NG, G) for t in tables128]
    ef32 = jnp.concatenate(
        [edge_feat_comment, jnp.ones((E, 1), _f32),
         jnp.zeros((E, G - CONV - 1), _f32)], axis=1)
    i32 = jnp.int32
    sums = _sc_segment_sums(tables, ef32,
                            edge_index_publish.astype(i32),
                            edge_index_comment.astype(i32),
                            edge_index_ucu.astype(i32))
    pub, com, ucu = _finalize(W_ep, b_ep, sums)
    return (pub, com, ucu)


# speedup vs baseline: 2.0952x; 1.0181x over previous
import functools

import jax
import jax.numpy as jnp
from jax import lax
from jax.experimental import pallas as pl
from jax.experimental.pallas import tpu as pltpu
from jax.experimental.pallas import tpu_sc as plsc

N_NODE = 50000
E = 160000
D = 128
G = 32
NG = D // G
CONV = 16

NS = 16
K = 80
BPW = 125
HB = BPW
EPW = K * BPW
EPAD = NS * EPW
NPAD = 50048
RPW = NPAD // NS
TRASH = N_NODE

_f32 = jnp.float32


def _tables_body(hu, uc, wp, bp, wc, bc, w1, w2, bv, lg, lb, *outs):
    x = hu[...]
    tp = jnp.dot(x, wp[...], preferred_element_type=_f32) + bp[...]
    tc = (jnp.dot(x, wc[...], preferred_element_type=_f32) + bc[...]) * 0.7
    z = (jnp.dot(x, w1[...], preferred_element_type=_f32)
         + jnp.dot(uc[...], w2[...], preferred_element_type=_f32) + bv[...])
    mu = jnp.mean(z, axis=-1, keepdims=True)
    var = jnp.mean((z - mu) ** 2, axis=-1, keepdims=True)
    nm = jnp.maximum((z - mu) / jnp.sqrt(var + 1e-5) * lg[...] + lb[...], 0.0)
    outs[0][...] = tp
    outs[1][...] = tc
    outs[2][...] = nm


def _node_tables(h_user, user_context, W_pub, b_pub, W_com, b_com,
                 W1, W2, b_conv, ln_g, ln_b):
    blk = 2048
    grid = ((N_NODE + blk - 1) // blk,)
    full = lambda r, c: pl.BlockSpec((r, c), lambda i: (0, 0))
    return pl.pallas_call(
        _tables_body,
        grid=grid,
        in_specs=[
            pl.BlockSpec((blk, D), lambda i: (i, 0)),
            pl.BlockSpec((blk, CONV), lambda i: (i, 0)),
            full(D, D), full(1, D), full(D, D), full(1, D),
            full(D, D), full(CONV, D), full(1, D), full(1, D), full(1, D),
        ],
        out_specs=[pl.BlockSpec((blk, D), lambda i: (i, 0))] * 3,
        out_shape=[jax.ShapeDtypeStruct((N_NODE, D), _f32)] * 3,
    )(h_user, user_context, W_pub, b_pub.reshape(1, D), W_com,
      b_com.reshape(1, D), W1, W2, b_conv.reshape(1, D),
      ln_g.reshape(1, D), ln_b.reshape(1, D))


def _sc_body(*refs):
    (tbls, ef32, eip, eic, eiu, opub, ocom, oucu, aux,
     acc, idxg, idxs, r0, r1, r2, cbuf, g0, g1, g2, s0, s1, s2, ss) = (
        refs[0:3], refs[3], refs[4], refs[5], refs[6], refs[7], refs[8],
        refs[9], refs[10], refs[11], refs[12], refs[13], refs[14], refs[15],
        refs[16], refs[17], refs[18], refs[19], refs[20], refs[21], refs[22],
        refs[23], refs[24])

    cid = lax.axis_index("c")
    sid = lax.axis_index("s")
    bufs = (r0, r1, r2)
    gsems = (g0, g1, g2)
    ssems = (s0, s1, s2)

    z16 = jnp.zeros((16,), _f32)
    one16 = jnp.where(lax.iota(jnp.int32, 16) == 0, 1.0, 0.0).astype(_f32)

    @pl.loop(0, K)
    def _(i):
        cbuf[i, pl.ds(0, 16)] = one16
        cbuf[i, pl.ds(16, 16)] = z16

    def one_pass(gather_tbl, grp, seq_tbl, eidx, out_hbm, col):
        @pl.loop(0, K)
        def _(i):
            r0[i, pl.ds(0, 16)] = z16
            r0[i, pl.ds(16, 16)] = z16
        base = sid * RPW
        for k in range(RPW // K):
            pltpu.async_copy(r0, acc.at[pl.ds(base + k * K, K)], ss)
        pltpu.async_copy(r0.at[pl.ds(0, RPW % K)],
                         acc.at[pl.ds(base + (RPW // K) * K, RPW % K)], ss)
        for k in range(RPW // K):
            pltpu.make_async_copy(r0, acc.at[pl.ds(base + k * K, K)],
                                  ss).wait()
        pltpu.make_async_copy(
            r0.at[pl.ds(0, RPW % K)],
            acc.at[pl.ds(base + (RPW // K) * K, RPW % K)], ss).wait()
        plsc.subcore_barrier()

        pltpu.sync_copy(eidx.at[1, pl.ds(sid * EPW, EPW)], idxs)
        if gather_tbl is not None:
            pltpu.sync_copy(eidx.at[0, pl.ds(sid * EPW, EPW)], idxg)

            @pl.loop(0, EPW // 16)
            def _(i):
                v = idxg[pl.ds(i * 16, 16)]
                idxg[pl.ds(i * 16, 16)] = v * 4 + grp

        if gather_tbl is None and seq_tbl is None:
            @pl.loop(0, BPW)
            def _(j):
                pltpu.async_copy(cbuf, acc.at[idxs.at[pl.ds(j * K, K)]],
                                 ss, add=True)

            @pl.loop(0, BPW)
            def _(j):
                pltpu.make_async_copy(cbuf, acc.at[idxs.at[pl.ds(j * K, K)]],
                                      ss).wait()
        else:
            def g_desc(j, b):
                if gather_tbl is not None:
                    return (gather_tbl.at[idxg.at[pl.ds(j * K, K)]],
                            bufs[b], gsems[b])
                return (seq_tbl.at[pl.ds(sid * EPW + j * K, K)],
                        bufs[b], gsems[b])

            def s_desc(j, b):
                return (bufs[b], acc.at[idxs.at[pl.ds(j * K, K)]], ssems[b])

            pltpu.async_copy(*g_desc(0, 0))
            pltpu.async_copy(*g_desc(1, 1))

            @pl.loop(0, BPW, step=3)
            def _(j0):
                for b in range(3):
                    j = j0 + b

                    @pl.when(j < BPW)
                    def _(j=j, b=b):
                        pltpu.make_async_copy(*g_desc(j, b)).wait()
                        pltpu.async_copy(*s_desc(j, b), add=True)

                        @pl.when(j >= 1)
                        def _(j=j, b=b):
                            pltpu.make_async_copy(
                                *s_desc(j - 1, (b + 2) % 3)).wait()

                        @pl.when(j + 2 < BPW)
                        def _(j=j, b=b):
                            pltpu.async_copy(*g_desc(j + 2, (b + 2) % 3))

            pltpu.make_async_copy(*s_desc(BPW - 1, (BPW - 1) % 3)).wait()

        plsc.subcore_barrier()
        pltpu.sync_copy(acc.at[pl.ds(sid * RPW, RPW)],
                        out_hbm.at[pl.ds(sid * RPW, RPW), pl.ds(col, G)])
        plsc.subcore_barrier()

    passes = []
    for g in range(NG):
        owner = 0 if g < 2 else 1
        passes.append((tbls[0], g, None, eip, opub, G * g, owner))
        passes.append((tbls[1], g, None, eic, ocom, G * g, owner))
        passes.append((tbls[2], g, None, eiu, oucu, G * g, owner))
    passes.append((None, 0, ef32, eic, aux, 0, 0))
    passes.append((None, 0, None, eip, aux, 32, 1))
    passes.append((None, 0, None, eiu, aux, 64, 1))

    for p, (gt, grp, st, e_h, o_h, col, owner) in enumerate(passes):
        kind = "gather" if gt is not None else ("seq" if st is not None else "cnt")
        with jax.named_scope(f"pass{p:02d}_{kind}_c{owner}"):
            @pl.when(cid == owner)
            def _(gt=gt, grp=grp, st=st, e_h=e_h, o_h=o_h, col=col):
                one_pass(gt, grp, st, e_h, o_h, col)


def _sc_segment_sums(tables, ef32, eip, eic, eiu):
    mesh = plsc.VectorSubcoreMesh(core_axis_name="c", subcore_axis_name="s")
    kern = pl.kernel(
        _sc_body,
        out_type=[jax.ShapeDtypeStruct((NPAD, D), _f32)] * 4,
        mesh=mesh,
        compiler_params=pltpu.CompilerParams(use_tc_tiling_on_sc=False),
        scratch_types=[
            pltpu.VMEM_SHARED((NPAD, G), _f32),
            pltpu.VMEM((EPW,), jnp.int32),
            pltpu.VMEM((EPW,), jnp.int32),
            pltpu.VMEM((K, G), _f32),
            pltpu.VMEM((K, G), _f32),
            pltpu.VMEM((K, G), _f32),
            pltpu.VMEM((K, G), _f32),
            pltpu.SemaphoreType.DMA,
            pltpu.SemaphoreType.DMA,
            pltpu.SemaphoreType.DMA,
            pltpu.SemaphoreType.DMA,
            pltpu.SemaphoreType.DMA,
            pltpu.SemaphoreType.DMA,
            pltpu.SemaphoreType.DMA,
        ],
    )
    return kern(*tables, ef32, eip, eic, eiu)


def _fin_body(wep, bep, spub, scom, sucu, aux, pub, com, ucu):
    a = aux[...]
    cnt_c = a[:, CONV:CONV + 1]
    invp = 1.0 / jnp.maximum(a[:, 32:33], 1.0)
    invc = 1.0 / jnp.maximum(cnt_c, 1.0)
    invu = 1.0 / jnp.maximum(a[:, 64:65], 1.0)
    pub[...] = spub[...] * invp
    nonzero = jnp.where(cnt_c >= 1.0, 0.3, 0.0)
    base = (jnp.dot(a[:, 0:CONV] * invc, wep[...],
                    preferred_element_type=_f32) + bep[...]) * nonzero
    com[...] = scom[...] * invc + base
    ucu[...] = sucu[...] * invu


def _finalize(W_ep, b_ep, sums):
    blk = 1024
    grid = ((N_NODE + blk - 1) // blk,)
    return pl.pallas_call(
        _fin_body,
        grid=grid,
        in_specs=[pl.BlockSpec((CONV, D), lambda i: (0, 0)),
                  pl.BlockSpec((1, D), lambda i: (0, 0))] +
                 [pl.BlockSpec((blk, D), lambda i: (i, 0))] * 4,
        out_specs=[pl.BlockSpec((blk, D), lambda i: (i, 0))] * 3,
        out_shape=[jax.ShapeDtypeStruct((N_NODE, D), _f32)] * 3,
    )(W_ep, b_ep.reshape(1, D), *sums)


@jax.jit
def kernel(h_user, h_post, user_context, edge_feat_comment, W_pub, b_pub,
           W_com, b_com, W_conv, b_conv, ln_g, ln_b, W_ep, b_ep,
           edge_index_publish, edge_index_comment, edge_index_ucu):
    tables128 = _node_tables(h_user, user_context, W_pub, b_pub, W_com, b_com,
                             W_conv[:D], W_conv[D:], b_conv, ln_g, ln_b)
    tables = [t.reshape(N_NODE * NG, G) for t in tables128]
    ef32 = jnp.concatenate(
        [edge_feat_comment, jnp.ones((E, 1), _f32),
         jnp.zeros((E, G - CONV - 1), _f32)], axis=1)
    i32 = jnp.int32
    sums = _sc_segment_sums(tables, ef32,
                            edge_index_publish.astype(i32),
                            edge_index_comment.astype(i32),
                            edge_index_ucu.astype(i32))
    pub, com, ucu = _finalize(W_ep, b_ep, sums)
    return (pub, com, ucu)


# speedup vs baseline: 2.1916x; 1.0460x over previous
import functools

import jax
import jax.numpy as jnp
from jax import lax
from jax.experimental import pallas as pl
from jax.experimental.pallas import tpu as pltpu
from jax.experimental.pallas import tpu_sc as plsc

N_NODE = 50000
E = 160000
D = 128
G = 32
NG = D // G
CONV = 16

NS = 16
K = 80
BPW = 125
HB = BPW
EPW = K * BPW
EPAD = NS * EPW
NPAD = 50048
RPW = NPAD // NS
TRASH = N_NODE

_f32 = jnp.float32


def _tables_body(hu, uc, wp, bp, wc, bc, w1, w2, bv, lg, lb, *outs):
    x = hu[...]
    tp = jnp.dot(x, wp[...], preferred_element_type=_f32) + bp[...]
    tc = (jnp.dot(x, wc[...], preferred_element_type=_f32) + bc[...]) * 0.7
    z = (jnp.dot(x, w1[...], preferred_element_type=_f32)
         + jnp.dot(uc[...], w2[...], preferred_element_type=_f32) + bv[...])
    mu = jnp.mean(z, axis=-1, keepdims=True)
    var = jnp.mean((z - mu) ** 2, axis=-1, keepdims=True)
    nm = jnp.maximum((z - mu) / jnp.sqrt(var + 1e-5) * lg[...] + lb[...], 0.0)
    outs[0][...] = tp
    outs[1][...] = tc
    outs[2][...] = nm


def _node_tables(h_user, user_context, W_pub, b_pub, W_com, b_com,
                 W1, W2, b_conv, ln_g, ln_b):
    blk = 2048
    grid = ((N_NODE + blk - 1) // blk,)
    full = lambda r, c: pl.BlockSpec((r, c), lambda i: (0, 0))
    return pl.pallas_call(
        _tables_body,
        grid=grid,
        in_specs=[
            pl.BlockSpec((blk, D), lambda i: (i, 0)),
            pl.BlockSpec((blk, CONV), lambda i: (i, 0)),
            full(D, D), full(1, D), full(D, D), full(1, D),
            full(D, D), full(CONV, D), full(1, D), full(1, D), full(1, D),
        ],
        out_specs=[pl.BlockSpec((blk, D), lambda i: (i, 0))] * 3,
        out_shape=[jax.ShapeDtypeStruct((N_NODE, D), _f32)] * 3,
    )(h_user, user_context, W_pub, b_pub.reshape(1, D), W_com,
      b_com.reshape(1, D), W1, W2, b_conv.reshape(1, D),
      ln_g.reshape(1, D), ln_b.reshape(1, D))


def _make_sc_body(n_in, build_passes):
    def _body(*refs):
        ins = refs[0:n_in]
        rest = refs[n_in:]
        outs = rest[:len(rest) - 14]
        (acc, idxg, idxs, r0, r1, r2, cbuf,
         g0, g1, g2, s0, s1, s2, ss) = rest[len(rest) - 14:]
        _sc_program(ins, outs, build_passes, acc, idxg, idxs, r0, r1, r2,
                    cbuf, (g0, g1, g2), (s0, s1, s2), ss)
    return _body


def _sc_program(ins, outs, build_passes, acc, idxg, idxs, r0, r1, r2,
                cbuf, gsems_t, ssems_t, ss):
    cid = lax.axis_index("c")
    sid = lax.axis_index("s")
    bufs = (r0, r1, r2)
    gsems = gsems_t
    ssems = ssems_t

    z16 = jnp.zeros((16,), _f32)
    one16 = jnp.where(lax.iota(jnp.int32, 16) == 0, 1.0, 0.0).astype(_f32)

    @pl.loop(0, K)
    def _(i):
        cbuf[i, pl.ds(0, 16)] = one16
        cbuf[i, pl.ds(16, 16)] = z16

    def one_pass(gather_tbl, grp, seq_tbl, eidx, out_hbm, col):
        @pl.loop(0, K)
        def _(i):
            r0[i, pl.ds(0, 16)] = z16
            r0[i, pl.ds(16, 16)] = z16
        base = sid * RPW
        for k in range(RPW // K):
            pltpu.async_copy(r0, acc.at[pl.ds(base + k * K, K)], ss)
        pltpu.async_copy(r0.at[pl.ds(0, RPW % K)],
                         acc.at[pl.ds(base + (RPW // K) * K, RPW % K)], ss)
        for k in range(RPW // K):
            pltpu.make_async_copy(r0, acc.at[pl.ds(base + k * K, K)],
                                  ss).wait()
        pltpu.make_async_copy(
            r0.at[pl.ds(0, RPW % K)],
            acc.at[pl.ds(base + (RPW // K) * K, RPW % K)], ss).wait()
        plsc.subcore_barrier()

        pltpu.sync_copy(eidx.at[1, pl.ds(sid * EPW, EPW)], idxs)
        if gather_tbl is not None:
            pltpu.sync_copy(eidx.at[0, pl.ds(sid * EPW, EPW)], idxg)

            @pl.loop(0, EPW // 16)
            def _(i):
                v = idxg[pl.ds(i * 16, 16)]
                idxg[pl.ds(i * 16, 16)] = v * 4 + grp

        if gather_tbl is None and seq_tbl is None:
            @pl.loop(0, BPW)
            def _(j):
                pltpu.async_copy(cbuf, acc.at[idxs.at[pl.ds(j * K, K)]],
                                 ss, add=True)

            @pl.loop(0, BPW)
            def _(j):
                pltpu.make_async_copy(cbuf, acc.at[idxs.at[pl.ds(j * K, K)]],
                                      ss).wait()
        else:
            def g_desc(j, b):
                if gather_tbl is not None:
                    return (gather_tbl.at[idxg.at[pl.ds(j * K, K)]],
                            bufs[b], gsems[b])
                return (seq_tbl.at[pl.ds(sid * EPW + j * K, K)],
                        bufs[b], gsems[b])

            def s_desc(j, b):
                return (bufs[b], acc.at[idxs.at[pl.ds(j * K, K)]], ssems[b])

            pltpu.async_copy(*g_desc(0, 0))
            pltpu.async_copy(*g_desc(1, 1))

            @pl.loop(0, BPW, step=3)
            def _(j0):
                for b in range(3):
                    j = j0 + b

                    @pl.when(j < BPW)
                    def _(j=j, b=b):
                        pltpu.make_async_copy(*g_desc(j, b)).wait()
                        pltpu.async_copy(*s_desc(j, b), add=True)

                        @pl.when(j >= 1)
                        def _(j=j, b=b):
                            pltpu.make_async_copy(
                                *s_desc(j - 1, (b + 2) % 3)).wait()

                        @pl.when(j + 2 < BPW)
                        def _(j=j, b=b):
                            pltpu.async_copy(*g_desc(j + 2, (b + 2) % 3))

            pltpu.make_async_copy(*s_desc(BPW - 1, (BPW - 1) % 3)).wait()

        plsc.subcore_barrier()
        pltpu.sync_copy(acc.at[pl.ds(sid * RPW, RPW)],
                        out_hbm.at[pl.ds(sid * RPW, RPW), pl.ds(col, G)])
        plsc.subcore_barrier()

    passes = build_passes(ins, outs)
    for p, (gt, grp, st, e_h, o_h, col, owner) in enumerate(passes):
        kind = "gather" if gt is not None else ("seq" if st is not None else "cnt")
        with jax.named_scope(f"pass{p:02d}_{kind}_c{owner}"):
            @pl.when(cid == owner)
            def _(gt=gt, grp=grp, st=st, e_h=e_h, o_h=o_h, col=col):
                one_pass(gt, grp, st, e_h, o_h, col)


def _sc_call(build_passes, n_out, args):
    mesh = plsc.VectorSubcoreMesh(core_axis_name="c", subcore_axis_name="s")
    kern = pl.kernel(
        _make_sc_body(len(args), build_passes),
        out_type=[jax.ShapeDtypeStruct((NPAD, D), _f32)] * n_out,
        mesh=mesh,
        compiler_params=pltpu.CompilerParams(use_tc_tiling_on_sc=False),
        scratch_types=[
            pltpu.VMEM_SHARED((NPAD, G), _f32),
            pltpu.VMEM((EPW,), jnp.int32),
            pltpu.VMEM((EPW,), jnp.int32),
            pltpu.VMEM((K, G), _f32),
            pltpu.VMEM((K, G), _f32),
            pltpu.VMEM((K, G), _f32),
            pltpu.VMEM((K, G), _f32),
            pltpu.SemaphoreType.DMA,
            pltpu.SemaphoreType.DMA,
            pltpu.SemaphoreType.DMA,
            pltpu.SemaphoreType.DMA,
            pltpu.SemaphoreType.DMA,
            pltpu.SemaphoreType.DMA,
            pltpu.SemaphoreType.DMA,
        ],
    )
    return kern(*args)


def _aux_passes(ins, outs):
    ef32, eip, eic, eiu = ins
    aux = outs[0]
    return [
        (None, 0, ef32, eic, aux, 0, 0),
        (None, 0, None, eip, aux, 32, 1),
        (None, 0, None, eiu, aux, 64, 1),
    ]


def _group_passes(ins, outs):
    t_pub, t_com, t_ucu, eip, eic, eiu = ins
    opub, ocom, oucu = outs
    passes = []
    for g in range(NG):
        owner = 0 if g < 2 else 1
        passes.append((t_pub, g, None, eip, opub, G * g, owner))
        passes.append((t_com, g, None, eic, ocom, G * g, owner))
        passes.append((t_ucu, g, None, eiu, oucu, G * g, owner))
    return passes


def _fin_body(wep, bep, spub, scom, sucu, aux, pub, com, ucu):
    a = aux[...]
    cnt_c = a[:, CONV:CONV + 1]
    invp = 1.0 / jnp.maximum(a[:, 32:33], 1.0)
    invc = 1.0 / jnp.maximum(cnt_c, 1.0)
    invu = 1.0 / jnp.maximum(a[:, 64:65], 1.0)
    pub[...] = spub[...] * invp
    nonzero = jnp.where(cnt_c >= 1.0, 0.3, 0.0)
    base = (jnp.dot(a[:, 0:CONV] * invc, wep[...],
                    preferred_element_type=_f32) + bep[...]) * nonzero
    com[...] = scom[...] * invc + base
    ucu[...] = sucu[...] * invu


def _finalize(W_ep, b_ep, sums):
    blk = 1024
    grid = ((N_NODE + blk - 1) // blk,)
    return pl.pallas_call(
        _fin_body,
        grid=grid,
        in_specs=[pl.BlockSpec((CONV, D), lambda i: (0, 0)),
                  pl.BlockSpec((1, D), lambda i: (0, 0))] +
                 [pl.BlockSpec((blk, D), lambda i: (i, 0))] * 4,
        out_specs=[pl.BlockSpec((blk, D), lambda i: (i, 0))] * 3,
        out_shape=[jax.ShapeDtypeStruct((N_NODE, D), _f32)] * 3,
    )(W_ep, b_ep.reshape(1, D), *sums)


@jax.jit
def kernel(h_user, h_post, user_context, edge_feat_comment, W_pub, b_pub,
           W_com, b_com, W_conv, b_conv, ln_g, ln_b, W_ep, b_ep,
           edge_index_publish, edge_index_comment, edge_index_ucu):
    tables128 = _node_tables(h_user, user_context, W_pub, b_pub, W_com, b_com,
                             W_conv[:D], W_conv[D:], b_conv, ln_g, ln_b)
    tables = [t.reshape(N_NODE * NG, G) for t in tables128]
    ef32 = jnp.concatenate(
        [edge_feat_comment, jnp.ones((E, 1), _f32),
         jnp.zeros((E, G - CONV - 1), _f32)], axis=1)
    i32 = jnp.int32
    eip = edge_index_publish.astype(i32)
    eic = edge_index_comment.astype(i32)
    eiu = edge_index_ucu.astype(i32)
    (aux,) = _sc_call(_aux_passes, 1, [ef32, eip, eic, eiu])
    sums = _sc_call(_group_passes, 3, [*tables, eip, eic, eiu])
    pub, com, ucu = _finalize(W_ep, b_ep, [*sums, aux])
    return (pub, com, ucu)


# speedup vs baseline: 2.5200x; 1.1498x over previous
import functools

import jax
import jax.numpy as jnp
from jax import lax
from jax.experimental import pallas as pl
from jax.experimental.pallas import tpu as pltpu
from jax.experimental.pallas import tpu_sc as plsc

N_NODE = 50000
E = 160000
D = 128
G = 32
NG = D // G
CONV = 16

NS = 16
K = 80
BPW = 125
HB = BPW
EPW = K * BPW
EPAD = NS * EPW
NPAD = 50048
RPW = NPAD // NS
TRASH = N_NODE

_f32 = jnp.float32


def _tables_body(hu, uc, wp, bp, wc, bc, w1, w2, bv, lg, lb, *outs):
    x = hu[...]
    tp = jnp.dot(x, wp[...], preferred_element_type=_f32) + bp[...]
    tc = (jnp.dot(x, wc[...], preferred_element_type=_f32) + bc[...]) * 0.7
    z = (jnp.dot(x, w1[...], preferred_element_type=_f32)
         + jnp.dot(uc[...], w2[...], preferred_element_type=_f32) + bv[...])
    mu = jnp.mean(z, axis=-1, keepdims=True)
    var = jnp.mean((z - mu) ** 2, axis=-1, keepdims=True)
    nm = jnp.maximum((z - mu) / jnp.sqrt(var + 1e-5) * lg[...] + lb[...], 0.0)
    outs[0][...] = tp
    outs[1][...] = tc
    outs[2][...] = nm


def _node_tables(h_user, user_context, W_pub, b_pub, W_com, b_com,
                 W1, W2, b_conv, ln_g, ln_b):
    blk = 2048
    grid = ((N_NODE + blk - 1) // blk,)
    full = lambda r, c: pl.BlockSpec((r, c), lambda i: (0, 0))
    return pl.pallas_call(
        _tables_body,
        grid=grid,
        in_specs=[
            pl.BlockSpec((blk, D), lambda i: (i, 0)),
            pl.BlockSpec((blk, CONV), lambda i: (i, 0)),
            full(D, D), full(1, D), full(D, D), full(1, D),
            full(D, D), full(CONV, D), full(1, D), full(1, D), full(1, D),
        ],
        out_specs=[pl.BlockSpec((blk, D), lambda i: (i, 0))] * 3,
        out_shape=[jax.ShapeDtypeStruct((N_NODE, D), _f32)] * 3,
    )(h_user, user_context, W_pub, b_pub.reshape(1, D), W_com,
      b_com.reshape(1, D), W1, W2, b_conv.reshape(1, D),
      ln_g.reshape(1, D), ln_b.reshape(1, D))


def _make_sc_body(n_in, build_passes):
    def _body(*refs):
        ins = refs[0:n_in]
        rest = refs[n_in:]
        outs = rest[:len(rest) - 14]
        (acc, idxg, idxs, r0, r1, r2, cbuf,
         g0, g1, g2, s0, s1, s2, ss) = rest[len(rest) - 14:]
        _sc_program(ins, outs, build_passes, acc, idxg, idxs, r0, r1, r2,
                    cbuf, (g0, g1, g2), (s0, s1, s2), ss)
    return _body


def _sc_program(ins, outs, build_passes, acc, idxg, idxs, r0, r1, r2,
                cbuf, gsems_t, ssems_t, ss):
    cid = lax.axis_index("c")
    sid = lax.axis_index("s")
    bufs = (r0, r1, r2)
    gsems = gsems_t
    ssems = ssems_t

    z16 = jnp.zeros((16,), _f32)
    one16 = jnp.where(lax.iota(jnp.int32, 16) == 0, 1.0, 0.0).astype(_f32)

    @pl.loop(0, K)
    def _(i):
        cbuf[i, pl.ds(0, 16)] = one16
        cbuf[i, pl.ds(16, 16)] = z16

    def one_pass(gather_tbl, grp, seq_tbl, eidx, out_hbm, col):
        @pl.loop(0, K)
        def _(i):
            r0[i, pl.ds(0, 16)] = z16
            r0[i, pl.ds(16, 16)] = z16
        base = sid * RPW
        for k in range(RPW // K):
            pltpu.async_copy(r0, acc.at[pl.ds(base + k * K, K)], ss)
        pltpu.async_copy(r0.at[pl.ds(0, RPW % K)],
                         acc.at[pl.ds(base + (RPW // K) * K, RPW % K)], ss)
        for k in range(RPW // K):
            pltpu.make_async_copy(r0, acc.at[pl.ds(base + k * K, K)],
                                  ss).wait()
        pltpu.make_async_copy(
            r0.at[pl.ds(0, RPW % K)],
            acc.at[pl.ds(base + (RPW // K) * K, RPW % K)], ss).wait()
        plsc.subcore_barrier()

        pltpu.sync_copy(eidx.at[1, pl.ds(sid * EPW, EPW)], idxs)
        if gather_tbl is not None:
            pltpu.sync_copy(eidx.at[0, pl.ds(sid * EPW, EPW)], idxg)

            @pl.loop(0, EPW // 16)
            def _(i):
                v = idxg[pl.ds(i * 16, 16)]
                idxg[pl.ds(i * 16, 16)] = v * 4 + grp

        if gather_tbl is None and seq_tbl is None:
            @pl.loop(0, BPW)
            def _(j):
                pltpu.async_copy(cbuf, acc.at[idxs.at[pl.ds(j * K, K)]],
                                 ss, add=True)

            @pl.loop(0, BPW)
            def _(j):
                pltpu.make_async_copy(cbuf, acc.at[idxs.at[pl.ds(j * K, K)]],
                                      ss).wait()
        else:
            def g_desc(j, b):
                if gather_tbl is not None:
                    return (gather_tbl.at[idxg.at[pl.ds(j * K, K)]],
                            bufs[b], gsems[b])
                return (seq_tbl.at[pl.ds(sid * EPW + j * K, K)],
                        bufs[b], gsems[b])

            def s_desc(j, b):
                return (bufs[b], acc.at[idxs.at[pl.ds(j * K, K)]], ssems[b])

            pltpu.async_copy(*g_desc(0, 0))
            pltpu.async_copy(*g_desc(1, 1))

            @pl.loop(0, BPW, step=3)
            def _(j0):
                for b in range(3):
                    j = j0 + b

                    @pl.when(j < BPW)
                    def _(j=j, b=b):
                        @pl.when(j >= 1)
                        def _(j=j, b=b):
                            pltpu.make_async_copy(
                                *s_desc(j - 1, (b + 2) % 3)).wait()

                        @pl.when(j + 2 < BPW)
                        def _(j=j, b=b):
                            pltpu.async_copy(*g_desc(j + 2, (b + 2) % 3))
                        pltpu.make_async_copy(*g_desc(j, b)).wait()
                        pltpu.async_copy(*s_desc(j, b), add=True)

            pltpu.make_async_copy(*s_desc(BPW - 1, (BPW - 1) % 3)).wait()

        plsc.subcore_barrier()
        pltpu.sync_copy(acc.at[pl.ds(sid * RPW, RPW)],
                        out_hbm.at[pl.ds(sid * RPW, RPW), pl.ds(col, G)])
        plsc.subcore_barrier()

    passes = build_passes(ins, outs)
    for p, (gt, grp, st, e_h, o_h, col, owner) in enumerate(passes):
        kind = "gather" if gt is not None else ("seq" if st is not None else "cnt")
        with jax.named_scope(f"pass{p:02d}_{kind}_c{owner}"):
            @pl.when(cid == owner)
            def _(gt=gt, grp=grp, st=st, e_h=e_h, o_h=o_h, col=col):
                one_pass(gt, grp, st, e_h, o_h, col)


def _sc_call(build_passes, n_out, args):
    mesh = plsc.VectorSubcoreMesh(core_axis_name="c", subcore_axis_name="s")
    kern = pl.kernel(
        _make_sc_body(len(args), build_passes),
        out_type=[jax.ShapeDtypeStruct((NPAD, D), _f32)] * n_out,
        mesh=mesh,
        compiler_params=pltpu.CompilerParams(use_tc_tiling_on_sc=False),
        scratch_types=[
            pltpu.VMEM_SHARED((NPAD, G), _f32),
            pltpu.VMEM((EPW,), jnp.int32),
            pltpu.VMEM((EPW,), jnp.int32),
            pltpu.VMEM((K, G), _f32),
            pltpu.VMEM((K, G), _f32),
            pltpu.VMEM((K, G), _f32),
            pltpu.VMEM((K, G), _f32),
            pltpu.SemaphoreType.DMA,
            pltpu.SemaphoreType.DMA,
            pltpu.SemaphoreType.DMA,
            pltpu.SemaphoreType.DMA,
            pltpu.SemaphoreType.DMA,
            pltpu.SemaphoreType.DMA,
            pltpu.SemaphoreType.DMA,
        ],
    )
    return kern(*args)


def _aux_passes(ins, outs):
    ef32, eip, eic, eiu = ins
    aux = outs[0]
    return [
        (None, 0, ef32, eic, aux, 0, 0),
        (None, 0, None, eip, aux, 32, 1),
        (None, 0, None, eiu, aux, 64, 1),
    ]


def _group_passes(ins, outs):
    t_pub, t_com, t_ucu, eip, eic, eiu = ins[:6]
    opub, ocom, oucu = outs
    passes = []
    for g in range(NG):
        owner = 0 if g < 2 else 1
        passes.append((t_pub, g, None, eip, opub, G * g, owner))
        passes.append((t_com, g, None, eic, ocom, G * g, owner))
        passes.append((t_ucu, g, None, eiu, oucu, G * g, owner))
    return passes


def _fin_body(wep, bep, spub, scom, sucu, aux, pub, com, ucu):
    a = aux[...]
    cnt_c = a[:, CONV:CONV + 1]
    invp = 1.0 / jnp.maximum(a[:, 32:33], 1.0)
    invc = 1.0 / jnp.maximum(cnt_c, 1.0)
    invu = 1.0 / jnp.maximum(a[:, 64:65], 1.0)
    pub[...] = spub[...] * invp
    nonzero = jnp.where(cnt_c >= 1.0, 0.3, 0.0)
    base = (jnp.dot(a[:, 0:CONV] * invc, wep[...],
                    preferred_element_type=_f32) + bep[...]) * nonzero
    com[...] = scom[...] * invc + base
    ucu[...] = sucu[...] * invu


def _finalize(W_ep, b_ep, sums):
    blk = 1024
    grid = ((N_NODE + blk - 1) // blk,)
    return pl.pallas_call(
        _fin_body,
        grid=grid,
        in_specs=[pl.BlockSpec((CONV, D), lambda i: (0, 0)),
                  pl.BlockSpec((1, D), lambda i: (0, 0))] +
                 [pl.BlockSpec((blk, D), lambda i: (i, 0))] * 4,
        out_specs=[pl.BlockSpec((blk, D), lambda i: (i, 0))] * 3,
        out_shape=[jax.ShapeDtypeStruct((N_NODE, D), _f32)] * 3,
    )(W_ep, b_ep.reshape(1, D), *sums)


@jax.jit
def kernel(h_user, h_post, user_context, edge_feat_comment, W_pub, b_pub,
           W_com, b_com, W_conv, b_conv, ln_g, ln_b, W_ep, b_ep,
           edge_index_publish, edge_index_comment, edge_index_ucu):
    tables128 = _node_tables(h_user, user_context, W_pub, b_pub, W_com, b_com,
                             W_conv[:D], W_conv[D:], b_conv, ln_g, ln_b)
    tables = [t.reshape(N_NODE * NG, G) for t in tables128]
    ef32 = jnp.concatenate(
        [edge_feat_comment, jnp.ones((E, 1), _f32),
         jnp.zeros((E, G - CONV - 1), _f32)], axis=1)
    i32 = jnp.int32
    eip = edge_index_publish.astype(i32)
    eic = edge_index_comment.astype(i32)
    eiu = edge_index_ucu.astype(i32)
    (aux,) = _sc_call(_aux_passes, 1, [ef32, eip, eic, eiu])
    sums = _sc_call(_group_passes, 3, [*tables, eip, eic, eiu, aux])
    pub, com, ucu = _finalize(W_ep, b_ep, [*sums, aux])
    return (pub, com, ucu)
